# baseline (speedup 1.0000x reference)
import functools

import jax
import jax.numpy as jnp
from jax import lax
from jax.experimental import pallas as pl
from jax.experimental.pallas import tpu as pltpu
from jax.experimental.pallas import tpu_sc as plsc

N = 10000
NPAD = 10240
DUMMY = N
NC, NS, L = 2, 16, 16
NW = NC * NS
CH = 128
RPT = NPAD // NS
D = 64


def _make_deg_kernel(nchunk):
    mesh = plsc.VectorSubcoreMesh(core_axis_name="c", subcore_axis_name="s")

    @functools.partial(
        pl.kernel,
        mesh=mesh,
        out_type=jax.ShapeDtypeStruct((2, NW, NPAD), jnp.float32),
        compiler_params=pltpu.CompilerParams(needs_layout_passes=False,
                                             use_tc_tiling_on_sc=False),
        scratch_types=[
            pltpu.VMEM((nchunk, CH), jnp.int32),
            pltpu.VMEM((NPAD,), jnp.float32),
        ],
    )
    def deg_kernel(d1_hbm, d2_hbm, out_hbm, dstv, hist):
        c = lax.axis_index("c")
        s = lax.axis_index("s")
        gid = c * NS + s
        zero16 = jnp.zeros((L,), jnp.float32)
        ones16 = jnp.ones((L,), jnp.float32)

        for l, d_hbm in ((0, d1_hbm), (1, d2_hbm)):
            pltpu.sync_copy(d_hbm.at[gid], dstv)

            def _zh(i, _):
                hist[pl.ds(i * L, L)] = zero16
                return _
            lax.fori_loop(0, NPAD // L, _zh, None)

            def _acc(r, _):
                for k in range(CH // L):
                    idx = dstv[r, pl.ds(k * L, L)]
                    plsc.addupdate_scatter(hist, [idx], ones16)
                return _
            lax.fori_loop(0, nchunk, _acc, None)
            pltpu.sync_copy(hist, out_hbm.at[l].at[gid])

    return deg_kernel


def _make_seg_kernel(nchunk):
    mesh = plsc.VectorSubcoreMesh(core_axis_name="c", subcore_axis_name="s")
    acc_t = jax.ShapeDtypeStruct((NC, NPAD, D), jnp.float32)

    @functools.partial(
        pl.kernel,
        mesh=mesh,
        out_type=[acc_t, acc_t],
        compiler_params=pltpu.CompilerParams(needs_layout_passes=False,
                                             use_tc_tiling_on_sc=False),
        scratch_types=[
            pltpu.VMEM((nchunk * CH,), jnp.int32),
            pltpu.VMEM((nchunk, CH), jnp.int32),
            pltpu.VMEM((2 * CH, D), jnp.float32),
            pltpu.VMEM((CH, D), jnp.float32),
            pltpu.VMEM_SHARED((NPAD, D), jnp.float32),
            pltpu.VMEM_SHARED((NPAD, D), jnp.float32),
            pltpu.SemaphoreType.DMA,
        ],
    )
    def seg_kernel(hs1_hbm, hs2_hbm, s1_hbm, d1_hbm, s2_hbm, d2_hbm,
                   o1_hbm, o2_hbm,
                   si, di, rows, bounce, acc1, acc2, semA):
        c = lax.axis_index("c")
        s = lax.axis_index("s")
        gid = c * NS + s
        zero16 = jnp.zeros((L,), jnp.float32)

        def _zb(r, _):
            for k in range(D // L):
                bounce[r, pl.ds(k * L, L)] = zero16
            return _
        lax.fori_loop(0, CH, _zb, None)
        for acc in (acc1, acc2):
            for q in range(RPT // CH):
                pltpu.sync_copy(bounce, acc.at[pl.ds(s * RPT + q * CH, CH)])
        plsc.subcore_barrier()

        for s_hbm, d_hbm, hs_hbm, acc in ((s1_hbm, d1_hbm, hs1_hbm, acc1),
                                          (s2_hbm, d2_hbm, hs2_hbm, acc2)):
            pltpu.sync_copy(s_hbm.at[gid], si)
            pltpu.sync_copy(d_hbm.at[gid], di)

            def _pair(i, _):
                pltpu.async_copy(hs_hbm.at[si.at[pl.ds(i * 2 * CH, 2 * CH)]],
                                 rows, semA).wait()
                for q in range(2):
                    pltpu.sync_copy(rows.at[pl.ds(q * CH, CH)],
                                    acc.at[di.at[2 * i + q]], add=True)
                return _
            lax.fori_loop(0, nchunk // 2, _pair, None)

        plsc.subcore_barrier()
        for acc, o_hbm in ((acc1, o1_hbm), (acc2, o2_hbm)):
            for q in range(RPT // CH):
                pltpu.sync_copy(acc.at[pl.ds(s * RPT + q * CH, CH)], bounce)
                pltpu.sync_copy(bounce,
                                o_hbm.at[c].at[pl.ds(s * RPT + q * CH, CH)])

    return seg_kernel


_B = 1000


def _dinv_pair(degp_blk):
    deg1 = jnp.sum(degp_blk[0], axis=1).reshape(_B, 1) + 1.0
    deg2 = jnp.sum(degp_blk[1], axis=1).reshape(_B, 1) + 1.0
    return lax.rsqrt(deg1), lax.rsqrt(deg2)


def _deg_spec():
    return pl.BlockSpec((2, _B, NW), lambda i: (0, i, 0))


def _stage1_body(x_ref, w11_ref, w21_ref, degp_ref, hs11_ref, hs21_ref):
    dinv1, dinv2 = _dinv_pair(degp_ref[...])
    x = x_ref[...]
    hs11_ref[...] = dinv1 * jnp.dot(x, w11_ref[...],
                                    preferred_element_type=jnp.float32)
    hs21_ref[...] = dinv2 * jnp.dot(x, w21_ref[...],
                                    preferred_element_type=jnp.float32)


def _elu(v):
    return jnp.where(v > 0, v, jnp.exp(jnp.minimum(v, 0.0)) - 1.0)


def _stage2_body(a11_ref, a21_ref, hs11_ref, hs21_ref, degp_ref,
                 b11_ref, b21_ref, wg1a_ref, wg1b_ref, bg1_ref,
                 w12_ref, w22_ref, hs12_ref, hs22_ref):
    dinv1, dinv2 = _dinv_pair(degp_ref[...])
    a11 = a11_ref[0] + a11_ref[1]
    a21 = a21_ref[0] + a21_ref[1]
    s11 = _elu(dinv1 * (a11 + hs11_ref[...]) + b11_ref[...])
    s21 = _elu(dinv2 * (a21 + hs21_ref[...]) + b21_ref[...])
    z = (jnp.sum(s11 * wg1a_ref[...], axis=1, keepdims=True)
         + jnp.sum(s21 * wg1b_ref[...], axis=1, keepdims=True) + bg1_ref[...])
    g = jax.nn.sigmoid(z)
    mid = g * s11 + (1.0 - g) * s21
    hs12_ref[...] = dinv1 * jnp.dot(mid, w12_ref[...],
                                    preferred_element_type=jnp.float32)
    hs22_ref[...] = dinv2 * jnp.dot(mid, w22_ref[...],
                                    preferred_element_type=jnp.float32)


def _stage3_body(a12_ref, a22_ref, hs12_ref, hs22_ref, degp_ref,
                 b12_ref, b22_ref, wg2a_ref, wg2b_ref, bg2_ref,
                 wt_ref, bt_ref, out_ref):
    dinv1, dinv2 = _dinv_pair(degp_ref[...])
    s12 = dinv1 * (a12_ref[0] + a12_ref[1] + hs12_ref[...]) + b12_ref[...]
    s22 = dinv2 * (a22_ref[0] + a22_ref[1] + hs22_ref[...]) + b22_ref[...]
    z = (jnp.sum(s12 * wg2a_ref[...], axis=1, keepdims=True)
         + jnp.sum(s22 * wg2b_ref[...], axis=1, keepdims=True) + bg2_ref[...])
    g = jax.nn.sigmoid(z)
    o = g * s12 + (1.0 - g) * s22
    o = jnp.dot(o, wt_ref[...], preferred_element_type=jnp.float32) + bt_ref[...]
    m = jnp.max(o, axis=1, keepdims=True)
    e = o - m
    out_ref[...] = e - jnp.log(jnp.sum(jnp.exp(e), axis=1, keepdims=True))


def _full(shape):
    nd = len(shape)
    return pl.BlockSpec(shape, lambda i, _nd=nd: (0,) * nd)


def _rows(width):
    return pl.BlockSpec((_B, width), lambda i: (i, 0))


def _acc_spec():
    return pl.BlockSpec((NC, _B, D), lambda i: (0, i, 0))


def kernel(node_feature, one_adj_list, two_adj_list, W11, b11, W12, b12,
           W21, b21, W22, b22, Wg1, bg1, Wg2, bg2, Wt, bt):
    n, d_in = node_feature.shape
    e = one_adj_list.shape[1]
    ept = -(-e // (NW * 2 * CH)) * 2 * CH
    nchunk = ept // CH
    pad = NW * ept - e

    def prep(adj):
        src = jnp.concatenate([adj[0], jnp.zeros((pad,), jnp.int32)])
        dst = jnp.concatenate([adj[1], jnp.full((pad,), DUMMY, jnp.int32)])
        return src.reshape(NW, nchunk * CH), dst.reshape(NW, nchunk, CH)

    s1r, d1r = prep(one_adj_list)
    s2r, d2r = prep(two_adj_list)

    deg_parts = _make_deg_kernel(nchunk)(d1r, d2r)
    degp = jnp.swapaxes(deg_parts, 1, 2)

    grid = n // _B
    hs_t = jax.ShapeDtypeStruct((n, D), jnp.float32)

    hs11, hs21 = pl.pallas_call(
        _stage1_body,
        grid=(grid,),
        in_specs=[_rows(d_in), _full((d_in, D)), _full((d_in, D)), _deg_spec()],
        out_specs=[_rows(D), _rows(D)],
        out_shape=[hs_t, hs_t],
    )(node_feature, W11, W21, degp)

    seg = _make_seg_kernel(nchunk)
    a11, a21 = seg(hs11, hs21, s1r, d1r, s2r, d2r)

    b11r = b11.reshape(1, D)
    b21r = b21.reshape(1, D)
    wg1a = Wg1[:D, 0].reshape(1, D)
    wg1b = Wg1[D:, 0].reshape(1, D)
    bg1r = bg1.reshape(1, 1)

    hs12, hs22 = pl.pallas_call(
        _stage2_body,
        grid=(grid,),
        in_specs=[_acc_spec(), _acc_spec(), _rows(D), _rows(D), _deg_spec(),
                  _full((1, D)), _full((1, D)), _full((1, D)), _full((1, D)),
                  _full((1, 1)), _full((D, D)), _full((D, D))],
        out_specs=[_rows(D), _rows(D)],
        out_shape=[hs_t, hs_t],
    )(a11, a21, hs11, hs21, degp, b11r, b21r, wg1a, wg1b, bg1r, W12, W22)

    a12, a22 = seg(hs12, hs22, s1r, d1r, s2r, d2r)

    b12r = b12.reshape(1, D)
    b22r = b22.reshape(1, D)
    wg2a = Wg2[:D, 0].reshape(1, D)
    wg2b = Wg2[D:, 0].reshape(1, D)
    bg2r = bg2.reshape(1, 1)
    btr = bt.reshape(1, D)

    out = pl.pallas_call(
        _stage3_body,
        grid=(grid,),
        in_specs=[_acc_spec(), _acc_spec(), _rows(D), _rows(D), _deg_spec(),
                  _full((1, D)), _full((1, D)), _full((1, D)), _full((1, D)),
                  _full((1, 1)), _full((D, D)), _full((1, D))],
        out_specs=_rows(D),
        out_shape=jax.ShapeDtypeStruct((n, D), jnp.float32),
    )(a12, a22, hs12, hs22, degp, b12r, b22r, wg2a, wg2b, bg2r, Wt, btr)

    return out


# speedup vs baseline: 2.1306x; 2.1306x over previous
import functools

import jax
import jax.numpy as jnp
from jax import lax
from jax.experimental import pallas as pl
from jax.experimental.pallas import tpu as pltpu
from jax.experimental.pallas import tpu_sc as plsc

N = 10000
NPAD = 10240
DUMMY = N
NC, NS, L = 2, 16, 16
NW = NC * NS
CH = 128
RPT = NPAD // NS
D = 64


def _make_deg_kernel(nchunk):
    mesh = plsc.VectorSubcoreMesh(core_axis_name="c", subcore_axis_name="s")

    @functools.partial(
        pl.kernel,
        mesh=mesh,
        out_type=jax.ShapeDtypeStruct((2, NW, NPAD), jnp.float32),
        compiler_params=pltpu.CompilerParams(needs_layout_passes=False,
                                             use_tc_tiling_on_sc=False),
        scratch_types=[
            pltpu.VMEM((nchunk, CH), jnp.int32),
            pltpu.VMEM((NPAD,), jnp.float32),
        ],
    )
    def deg_kernel(d1_hbm, d2_hbm, out_hbm, dstv, hist):
        c = lax.axis_index("c")
        s = lax.axis_index("s")
        gid = c * NS + s
        zero16 = jnp.zeros((L,), jnp.float32)
        ones16 = jnp.ones((L,), jnp.float32)

        for l, d_hbm in ((0, d1_hbm), (1, d2_hbm)):
            pltpu.sync_copy(d_hbm.at[gid], dstv)

            def _zh(i, _):
                hist[pl.ds(i * L, L)] = zero16
                return _
            lax.fori_loop(0, NPAD // L, _zh, None)

            def _acc(r, _):
                for k in range(CH // L):
                    idx = dstv[r, pl.ds(k * L, L)]
                    plsc.addupdate_scatter(hist, [idx], ones16)
                return _
            lax.fori_loop(0, nchunk, _acc, None)
            pltpu.sync_copy(hist, out_hbm.at[l].at[gid])

    return deg_kernel


def _make_seg_kernel(nchunk):
    mesh = plsc.VectorSubcoreMesh(core_axis_name="c", subcore_axis_name="s")
    acc_t = jax.ShapeDtypeStruct((NC, NPAD, D), jnp.float32)
    nstage = N // NS

    @functools.partial(
        pl.kernel,
        mesh=mesh,
        out_type=acc_t,
        compiler_params=pltpu.CompilerParams(needs_layout_passes=False,
                                             use_tc_tiling_on_sc=False),
        scratch_types=[
            pltpu.VMEM((nchunk, CH), jnp.int32),
            pltpu.VMEM((nchunk, CH), jnp.int32),
            pltpu.VMEM((CH, D), jnp.float32),
            pltpu.VMEM((CH, D), jnp.float32),
            pltpu.VMEM_SHARED((N, D), jnp.float32),
            pltpu.VMEM_SHARED((NPAD, D), jnp.float32),
            pltpu.SemaphoreType.DMA,
        ],
    )
    def seg_kernel(hs_hbm, s_hbm, d_hbm, o_hbm,
                   si, di, rows, bounce, hs_sh, acc, semA):
        c = lax.axis_index("c")
        s = lax.axis_index("s")
        gid = c * NS + s
        zero16 = jnp.zeros((L,), jnp.float32)

        pltpu.sync_copy(hs_hbm.at[pl.ds(s * nstage, nstage)],
                        hs_sh.at[pl.ds(s * nstage, nstage)])

        def _zb(r, _):
            for k in range(D // L):
                bounce[r, pl.ds(k * L, L)] = zero16
            return _
        lax.fori_loop(0, CH, _zb, None)
        for q in range(RPT // CH):
            pltpu.sync_copy(bounce, acc.at[pl.ds(s * RPT + q * CH, CH)])

        pltpu.sync_copy(s_hbm.at[gid], si)
        pltpu.sync_copy(d_hbm.at[gid], di)
        plsc.subcore_barrier()

        def _chunk(j, _):
            pltpu.async_copy(hs_sh.at[si.at[j]], rows, semA).wait()
            pltpu.sync_copy(rows, acc.at[di.at[j]], add=True)
            return _
        lax.fori_loop(0, nchunk, _chunk, None)

        plsc.subcore_barrier()
        for q in range(RPT // CH):
            pltpu.sync_copy(acc.at[pl.ds(s * RPT + q * CH, CH)], bounce)
            pltpu.sync_copy(bounce,
                            o_hbm.at[c].at[pl.ds(s * RPT + q * CH, CH)])

    return seg_kernel


_B = 1000


def _dinv_pair(degp_blk):
    deg1 = jnp.sum(degp_blk[0], axis=1).reshape(_B, 1) + 1.0
    deg2 = jnp.sum(degp_blk[1], axis=1).reshape(_B, 1) + 1.0
    return lax.rsqrt(deg1), lax.rsqrt(deg2)


def _deg_spec():
    return pl.BlockSpec((2, _B, NW), lambda i: (0, i, 0))


def _stage1_body(x_ref, w11_ref, w21_ref, degp_ref, hs11_ref, hs21_ref):
    dinv1, dinv2 = _dinv_pair(degp_ref[...])
    x = x_ref[...]
    hs11_ref[...] = dinv1 * jnp.dot(x, w11_ref[...],
                                    preferred_element_type=jnp.float32)
    hs21_ref[...] = dinv2 * jnp.dot(x, w21_ref[...],
                                    preferred_element_type=jnp.float32)


def _elu(v):
    return jnp.where(v > 0, v, jnp.exp(jnp.minimum(v, 0.0)) - 1.0)


def _stage2_body(a11_ref, a21_ref, hs11_ref, hs21_ref, degp_ref,
                 b11_ref, b21_ref, wg1a_ref, wg1b_ref, bg1_ref,
                 w12_ref, w22_ref, hs12_ref, hs22_ref):
    dinv1, dinv2 = _dinv_pair(degp_ref[...])
    a11 = a11_ref[0] + a11_ref[1]
    a21 = a21_ref[0] + a21_ref[1]
    s11 = _elu(dinv1 * (a11 + hs11_ref[...]) + b11_ref[...])
    s21 = _elu(dinv2 * (a21 + hs21_ref[...]) + b21_ref[...])
    z = (jnp.sum(s11 * wg1a_ref[...], axis=1, keepdims=True)
         + jnp.sum(s21 * wg1b_ref[...], axis=1, keepdims=True) + bg1_ref[...])
    g = jax.nn.sigmoid(z)
    mid = g * s11 + (1.0 - g) * s21
    hs12_ref[...] = dinv1 * jnp.dot(mid, w12_ref[...],
                                    preferred_element_type=jnp.float32)
    hs22_ref[...] = dinv2 * jnp.dot(mid, w22_ref[...],
                                    preferred_element_type=jnp.float32)


def _stage3_body(a12_ref, a22_ref, hs12_ref, hs22_ref, degp_ref,
                 b12_ref, b22_ref, wg2a_ref, wg2b_ref, bg2_ref,
                 wt_ref, bt_ref, out_ref):
    dinv1, dinv2 = _dinv_pair(degp_ref[...])
    s12 = dinv1 * (a12_ref[0] + a12_ref[1] + hs12_ref[...]) + b12_ref[...]
    s22 = dinv2 * (a22_ref[0] + a22_ref[1] + hs22_ref[...]) + b22_ref[...]
    z = (jnp.sum(s12 * wg2a_ref[...], axis=1, keepdims=True)
         + jnp.sum(s22 * wg2b_ref[...], axis=1, keepdims=True) + bg2_ref[...])
    g = jax.nn.sigmoid(z)
    o = g * s12 + (1.0 - g) * s22
    o = jnp.dot(o, wt_ref[...], preferred_element_type=jnp.float32) + bt_ref[...]
    m = jnp.max(o, axis=1, keepdims=True)
    e = o - m
    out_ref[...] = e - jnp.log(jnp.sum(jnp.exp(e), axis=1, keepdims=True))


def _full(shape):
    nd = len(shape)
    return pl.BlockSpec(shape, lambda i, _nd=nd: (0,) * nd)


def _rows(width):
    return pl.BlockSpec((_B, width), lambda i: (i, 0))


def _acc_spec():
    return pl.BlockSpec((NC, _B, D), lambda i: (0, i, 0))


def kernel(node_feature, one_adj_list, two_adj_list, W11, b11, W12, b12,
           W21, b21, W22, b22, Wg1, bg1, Wg2, bg2, Wt, bt):
    n, d_in = node_feature.shape
    e = one_adj_list.shape[1]
    ept = -(-e // (NW * 2 * CH)) * 2 * CH
    nchunk = ept // CH
    pad = NW * ept - e

    def prep(adj):
        src = jnp.concatenate([adj[0], jnp.zeros((pad,), jnp.int32)])
        dst = jnp.concatenate([adj[1], jnp.full((pad,), DUMMY, jnp.int32)])
        return src.reshape(NW, nchunk, CH), dst.reshape(NW, nchunk, CH)

    s1r, d1r = prep(one_adj_list)
    s2r, d2r = prep(two_adj_list)

    deg_parts = _make_deg_kernel(nchunk)(d1r, d2r)
    degp = jnp.swapaxes(deg_parts, 1, 2)

    grid = n // _B
    hs_t = jax.ShapeDtypeStruct((n, D), jnp.float32)

    hs11, hs21 = pl.pallas_call(
        _stage1_body,
        grid=(grid,),
        in_specs=[_rows(d_in), _full((d_in, D)), _full((d_in, D)), _deg_spec()],
        out_specs=[_rows(D), _rows(D)],
        out_shape=[hs_t, hs_t],
    )(node_feature, W11, W21, degp)

    seg = _make_seg_kernel(nchunk)
    a11 = seg(hs11, s1r, d1r)
    a21 = seg(hs21, s2r, d2r)

    b11r = b11.reshape(1, D)
    b21r = b21.reshape(1, D)
    wg1a = Wg1[:D, 0].reshape(1, D)
    wg1b = Wg1[D:, 0].reshape(1, D)
    bg1r = bg1.reshape(1, 1)

    hs12, hs22 = pl.pallas_call(
        _stage2_body,
        grid=(grid,),
        in_specs=[_acc_spec(), _acc_spec(), _rows(D), _rows(D), _deg_spec(),
                  _full((1, D)), _full((1, D)), _full((1, D)), _full((1, D)),
                  _full((1, 1)), _full((D, D)), _full((D, D))],
        out_specs=[_rows(D), _rows(D)],
        out_shape=[hs_t, hs_t],
    )(a11, a21, hs11, hs21, degp, b11r, b21r, wg1a, wg1b, bg1r, W12, W22)

    a12 = seg(hs12, s1r, d1r)
    a22 = seg(hs22, s2r, d2r)

    b12r = b12.reshape(1, D)
    b22r = b22.reshape(1, D)
    wg2a = Wg2[:D, 0].reshape(1, D)
    wg2b = Wg2[D:, 0].reshape(1, D)
    bg2r = bg2.reshape(1, 1)
    btr = bt.reshape(1, D)

    out = pl.pallas_call(
        _stage3_body,
        grid=(grid,),
        in_specs=[_acc_spec(), _acc_spec(), _rows(D), _rows(D), _deg_spec(),
                  _full((1, D)), _full((1, D)), _full((1, D)), _full((1, D)),
                  _full((1, 1)), _full((D, D)), _full((1, D))],
        out_specs=_rows(D),
        out_shape=jax.ShapeDtypeStruct((n, D), jnp.float32),
    )(a12, a22, hs12, hs22, degp, b12r, b22r, wg2a, wg2b, bg2r, Wt, btr)

    return out


# speedup vs baseline: 2.6680x; 1.2522x over previous
import functools

import jax
import jax.numpy as jnp
from jax import lax
from jax.experimental import pallas as pl
from jax.experimental.pallas import tpu as pltpu
from jax.experimental.pallas import tpu_sc as plsc

N = 10000
NPAD = 10240
DUMMY = N
NC, NS, L = 2, 16, 16
NW = NC * NS
CH = 128
RPT = NPAD // NS
D = 64


def _make_deg_kernel(nchunk):
    mesh = plsc.VectorSubcoreMesh(core_axis_name="c", subcore_axis_name="s")

    @functools.partial(
        pl.kernel,
        mesh=mesh,
        out_type=jax.ShapeDtypeStruct((2, NW, NPAD), jnp.float32),
        compiler_params=pltpu.CompilerParams(needs_layout_passes=False,
                                             use_tc_tiling_on_sc=False),
        scratch_types=[
            pltpu.VMEM((nchunk, CH), jnp.int32),
            pltpu.VMEM((NPAD,), jnp.float32),
        ],
    )
    def deg_kernel(d1_hbm, d2_hbm, out_hbm, dstv, hist):
        c = lax.axis_index("c")
        s = lax.axis_index("s")
        gid = c * NS + s
        zero16 = jnp.zeros((L,), jnp.float32)
        ones16 = jnp.ones((L,), jnp.float32)

        for l, d_hbm in ((0, d1_hbm), (1, d2_hbm)):
            pltpu.sync_copy(d_hbm.at[gid], dstv)

            def _zh(i, _):
                hist[pl.ds(i * L, L)] = zero16
                return _
            lax.fori_loop(0, NPAD // L, _zh, None)

            def _acc(r, _):
                for k in range(CH // L):
                    idx = dstv[r, pl.ds(k * L, L)]
                    plsc.addupdate_scatter(hist, [idx], ones16)
                return _
            lax.fori_loop(0, nchunk, _acc, None)
            pltpu.sync_copy(hist, out_hbm.at[l].at[gid])

    return deg_kernel


def _make_seg_kernel(nchunk):
    mesh = plsc.VectorSubcoreMesh(core_axis_name="c", subcore_axis_name="s")
    acc_t = jax.ShapeDtypeStruct((NC, NPAD, D), jnp.float32)
    nstage = N // NS

    @functools.partial(
        pl.kernel,
        mesh=mesh,
        out_type=acc_t,
        compiler_params=pltpu.CompilerParams(needs_layout_passes=False,
                                             use_tc_tiling_on_sc=False),
        scratch_types=[
            pltpu.VMEM((nchunk, CH), jnp.int32),
            pltpu.VMEM((nchunk, CH), jnp.int32),
            pltpu.VMEM((CH, D), jnp.float32),
            pltpu.VMEM((CH, D), jnp.float32),
            pltpu.VMEM((CH, D), jnp.float32),
            pltpu.VMEM_SHARED((N, D), jnp.float32),
            pltpu.VMEM_SHARED((NPAD, D), jnp.float32),
            pltpu.SemaphoreType.DMA,
            pltpu.SemaphoreType.DMA,
        ],
    )
    def seg_kernel(hs_hbm, s_hbm, d_hbm, o_hbm,
                   si, di, rowsA, rowsB, bounce, hs_sh, acc, semA, semB):
        c = lax.axis_index("c")
        s = lax.axis_index("s")
        gid = c * NS + s
        zero16 = jnp.zeros((L,), jnp.float32)

        pltpu.sync_copy(hs_hbm.at[pl.ds(s * nstage, nstage)],
                        hs_sh.at[pl.ds(s * nstage, nstage)])

        def _zb(r, _):
            for k in range(D // L):
                bounce[r, pl.ds(k * L, L)] = zero16
            return _
        lax.fori_loop(0, CH, _zb, None)
        for q in range(RPT // CH):
            pltpu.sync_copy(bounce, acc.at[pl.ds(s * RPT + q * CH, CH)])

        pltpu.sync_copy(s_hbm.at[gid], si)
        pltpu.sync_copy(d_hbm.at[gid], di)
        plsc.subcore_barrier()

        bufs = ((rowsA, semA), (rowsB, semB))
        for b, (buf, sem) in enumerate(bufs):
            pltpu.async_copy(hs_sh.at[si.at[b]], buf, sem)

        def _pair(i, _):
            j = 2 * i
            for b, (buf, sem) in enumerate(bufs):
                pltpu.make_async_copy(hs_sh.at[si.at[j + b]], buf, sem).wait()
                pltpu.sync_copy(buf, acc.at[di.at[j + b]], add=True)
                pltpu.async_copy(hs_sh.at[si.at[j + b + 2]], buf, sem)
            return _
        lax.fori_loop(0, nchunk // 2 - 1, _pair, None)

        jlast = nchunk - 2
        for b, (buf, sem) in enumerate(bufs):
            pltpu.make_async_copy(hs_sh.at[si.at[jlast + b]], buf, sem).wait()
            pltpu.sync_copy(buf, acc.at[di.at[jlast + b]], add=True)

        plsc.subcore_barrier()
        for q in range(RPT // CH):
            pltpu.sync_copy(acc.at[pl.ds(s * RPT + q * CH, CH)], bounce)
            pltpu.sync_copy(bounce,
                            o_hbm.at[c].at[pl.ds(s * RPT + q * CH, CH)])

    return seg_kernel


_B = 1000


def _dinv_pair(degp_blk):
    deg1 = jnp.sum(degp_blk[0], axis=1).reshape(_B, 1) + 1.0
    deg2 = jnp.sum(degp_blk[1], axis=1).reshape(_B, 1) + 1.0
    return lax.rsqrt(deg1), lax.rsqrt(deg2)


def _deg_spec():
    return pl.BlockSpec((2, _B, NW), lambda i: (0, i, 0))


def _stage1_body(x_ref, w11_ref, w21_ref, degp_ref, hs11_ref, hs21_ref):
    dinv1, dinv2 = _dinv_pair(degp_ref[...])
    x = x_ref[...]
    hs11_ref[...] = dinv1 * jnp.dot(x, w11_ref[...],
                                    preferred_element_type=jnp.float32)
    hs21_ref[...] = dinv2 * jnp.dot(x, w21_ref[...],
                                    preferred_element_type=jnp.float32)


def _elu(v):
    return jnp.where(v > 0, v, jnp.exp(jnp.minimum(v, 0.0)) - 1.0)


def _stage2_body(a11_ref, a21_ref, hs11_ref, hs21_ref, degp_ref,
                 b11_ref, b21_ref, wg1a_ref, wg1b_ref, bg1_ref,
                 w12_ref, w22_ref, hs12_ref, hs22_ref):
    dinv1, dinv2 = _dinv_pair(degp_ref[...])
    a11 = a11_ref[0] + a11_ref[1]
    a21 = a21_ref[0] + a21_ref[1]
    s11 = _elu(dinv1 * (a11 + hs11_ref[...]) + b11_ref[...])
    s21 = _elu(dinv2 * (a21 + hs21_ref[...]) + b21_ref[...])
    z = (jnp.sum(s11 * wg1a_ref[...], axis=1, keepdims=True)
         + jnp.sum(s21 * wg1b_ref[...], axis=1, keepdims=True) + bg1_ref[...])
    g = jax.nn.sigmoid(z)
    mid = g * s11 + (1.0 - g) * s21
    hs12_ref[...] = dinv1 * jnp.dot(mid, w12_ref[...],
                                    preferred_element_type=jnp.float32)
    hs22_ref[...] = dinv2 * jnp.dot(mid, w22_ref[...],
                                    preferred_element_type=jnp.float32)


def _stage3_body(a12_ref, a22_ref, hs12_ref, hs22_ref, degp_ref,
                 b12_ref, b22_ref, wg2a_ref, wg2b_ref, bg2_ref,
                 wt_ref, bt_ref, out_ref):
    dinv1, dinv2 = _dinv_pair(degp_ref[...])
    s12 = dinv1 * (a12_ref[0] + a12_ref[1] + hs12_ref[...]) + b12_ref[...]
    s22 = dinv2 * (a22_ref[0] + a22_ref[1] + hs22_ref[...]) + b22_ref[...]
    z = (jnp.sum(s12 * wg2a_ref[...], axis=1, keepdims=True)
         + jnp.sum(s22 * wg2b_ref[...], axis=1, keepdims=True) + bg2_ref[...])
    g = jax.nn.sigmoid(z)
    o = g * s12 + (1.0 - g) * s22
    o = jnp.dot(o, wt_ref[...], preferred_element_type=jnp.float32) + bt_ref[...]
    m = jnp.max(o, axis=1, keepdims=True)
    e = o - m
    out_ref[...] = e - jnp.log(jnp.sum(jnp.exp(e), axis=1, keepdims=True))


def _full(shape):
    nd = len(shape)
    return pl.BlockSpec(shape, lambda i, _nd=nd: (0,) * nd)


def _rows(width):
    return pl.BlockSpec((_B, width), lambda i: (i, 0))


def _acc_spec():
    return pl.BlockSpec((NC, _B, D), lambda i: (0, i, 0))


def kernel(node_feature, one_adj_list, two_adj_list, W11, b11, W12, b12,
           W21, b21, W22, b22, Wg1, bg1, Wg2, bg2, Wt, bt):
    n, d_in = node_feature.shape
    e = one_adj_list.shape[1]
    ept = -(-e // (NW * 2 * CH)) * 2 * CH
    nchunk = ept // CH
    pad = NW * ept - e

    def prep(adj):
        src = jnp.concatenate([adj[0], jnp.zeros((pad,), jnp.int32)])
        dst = jnp.concatenate([adj[1], jnp.full((pad,), DUMMY, jnp.int32)])
        return src.reshape(NW, nchunk, CH), dst.reshape(NW, nchunk, CH)

    s1r, d1r = prep(one_adj_list)
    s2r, d2r = prep(two_adj_list)

    deg_parts = _make_deg_kernel(nchunk)(d1r, d2r)
    degp = jnp.swapaxes(deg_parts, 1, 2)

    grid = n // _B
    hs_t = jax.ShapeDtypeStruct((n, D), jnp.float32)

    hs11, hs21 = pl.pallas_call(
        _stage1_body,
        grid=(grid,),
        in_specs=[_rows(d_in), _full((d_in, D)), _full((d_in, D)), _deg_spec()],
        out_specs=[_rows(D), _rows(D)],
        out_shape=[hs_t, hs_t],
    )(node_feature, W11, W21, degp)

    seg = _make_seg_kernel(nchunk)
    a11 = seg(hs11, s1r, d1r)
    a21 = seg(hs21, s2r, d2r)

    b11r = b11.reshape(1, D)
    b21r = b21.reshape(1, D)
    wg1a = Wg1[:D, 0].reshape(1, D)
    wg1b = Wg1[D:, 0].reshape(1, D)
    bg1r = bg1.reshape(1, 1)

    hs12, hs22 = pl.pallas_call(
        _stage2_body,
        grid=(grid,),
        in_specs=[_acc_spec(), _acc_spec(), _rows(D), _rows(D), _deg_spec(),
                  _full((1, D)), _full((1, D)), _full((1, D)), _full((1, D)),
                  _full((1, 1)), _full((D, D)), _full((D, D))],
        out_specs=[_rows(D), _rows(D)],
        out_shape=[hs_t, hs_t],
    )(a11, a21, hs11, hs21, degp, b11r, b21r, wg1a, wg1b, bg1r, W12, W22)

    a12 = seg(hs12, s1r, d1r)
    a22 = seg(hs22, s2r, d2r)

    b12r = b12.reshape(1, D)
    b22r = b22.reshape(1, D)
    wg2a = Wg2[:D, 0].reshape(1, D)
    wg2b = Wg2[D:, 0].reshape(1, D)
    bg2r = bg2.reshape(1, 1)
    btr = bt.reshape(1, D)

    out = pl.pallas_call(
        _stage3_body,
        grid=(grid,),
        in_specs=[_acc_spec(), _acc_spec(), _rows(D), _rows(D), _deg_spec(),
                  _full((1, D)), _full((1, D)), _full((1, D)), _full((1, D)),
                  _full((1, 1)), _full((D, D)), _full((1, D))],
        out_specs=_rows(D),
        out_shape=jax.ShapeDtypeStruct((n, D), jnp.float32),
    )(a12, a22, hs12, hs22, degp, b12r, b22r, wg2a, wg2b, bg2r, Wt, btr)

    return out


# speedup vs baseline: 2.9407x; 1.1022x over previous
import functools

import jax
import jax.numpy as jnp
from jax import lax
from jax.experimental import pallas as pl
from jax.experimental.pallas import tpu as pltpu
from jax.experimental.pallas import tpu_sc as plsc

N = 10000
NPAD = 10240
DUMMY = N
NC, NS, L = 2, 16, 16
NW = NC * NS
CH = 128
RPT = NPAD // NS
D = 64


def _make_deg_kernel(nchunk):
    mesh = plsc.VectorSubcoreMesh(core_axis_name="c", subcore_axis_name="s")

    @functools.partial(
        pl.kernel,
        mesh=mesh,
        out_type=jax.ShapeDtypeStruct((2, NW, NPAD), jnp.float32),
        compiler_params=pltpu.CompilerParams(needs_layout_passes=False,
                                             use_tc_tiling_on_sc=False),
        scratch_types=[
            pltpu.VMEM((nchunk, CH), jnp.int32),
            pltpu.VMEM((NPAD,), jnp.float32),
        ],
    )
    def deg_kernel(d1_hbm, d2_hbm, out_hbm, dstv, hist):
        c = lax.axis_index("c")
        s = lax.axis_index("s")
        gid = c * NS + s
        zero16 = jnp.zeros((L,), jnp.float32)
        ones16 = jnp.ones((L,), jnp.float32)

        for l, d_hbm in ((0, d1_hbm), (1, d2_hbm)):
            pltpu.sync_copy(d_hbm.at[gid], dstv)

            def _zh(i, _):
                hist[pl.ds(i * L, L)] = zero16
                return _
            lax.fori_loop(0, NPAD // L, _zh, None)

            def _acc(r, _):
                for k in range(CH // L):
                    idx = dstv[r, pl.ds(k * L, L)]
                    plsc.addupdate_scatter(hist, [idx], ones16)
                return _
            lax.fori_loop(0, nchunk, _acc, None)
            pltpu.sync_copy(hist, out_hbm.at[l].at[gid])

    return deg_kernel


def _make_seg_kernel(nchunk):
    mesh = plsc.VectorSubcoreMesh(core_axis_name="c", subcore_axis_name="s")
    acc_t = jax.ShapeDtypeStruct((NC, NPAD, D), jnp.float32)
    nstage = N // NS

    @functools.partial(
        pl.kernel,
        mesh=mesh,
        out_type=acc_t,
        compiler_params=pltpu.CompilerParams(needs_layout_passes=False,
                                             use_tc_tiling_on_sc=False),
        scratch_types=[
            pltpu.VMEM((nchunk, CH), jnp.int32),
            pltpu.VMEM((nchunk, CH), jnp.int32),
            pltpu.VMEM((CH, D), jnp.float32),
            pltpu.VMEM((CH, D), jnp.float32),
            pltpu.VMEM((CH, D), jnp.float32),
            pltpu.VMEM_SHARED((N, D), jnp.float32),
            pltpu.VMEM_SHARED((NPAD, D), jnp.float32),
            pltpu.SemaphoreType.DMA,
            pltpu.SemaphoreType.DMA,
            pltpu.SemaphoreType.DMA,
        ],
    )
    def seg_kernel(hs_hbm, s_hbm, d_hbm, o_hbm,
                   si, di, rowsA, rowsB, bounce, hs_sh, acc, semA, semB, semS):
        c = lax.axis_index("c")
        s = lax.axis_index("s")
        gid = c * NS + s
        zero16 = jnp.zeros((L,), jnp.float32)

        pltpu.sync_copy(hs_hbm.at[pl.ds(s * nstage, nstage)],
                        hs_sh.at[pl.ds(s * nstage, nstage)])

        def _zb(r, _):
            for k in range(D // L):
                bounce[r, pl.ds(k * L, L)] = zero16
            return _
        lax.fori_loop(0, CH, _zb, None)
        for q in range(RPT // CH):
            pltpu.sync_copy(bounce, acc.at[pl.ds(s * RPT + q * CH, CH)])

        pltpu.sync_copy(s_hbm.at[gid], si)
        pltpu.sync_copy(d_hbm.at[gid], di)
        plsc.subcore_barrier()

        bufs = ((rowsA, semA), (rowsB, semB))
        for b, (buf, sem) in enumerate(bufs):
            pltpu.async_copy(hs_sh.at[si.at[b]], buf, sem)

        def _pair(i, _):
            j = 2 * i
            for b, (buf, sem) in enumerate(bufs):
                pltpu.make_async_copy(hs_sh.at[si.at[j + b]], buf, sem).wait()
                pltpu.async_copy(buf, acc.at[di.at[j + b]], semS, add=True)
                pltpu.async_copy(hs_sh.at[si.at[j + b + 2]], buf, sem)
            return _
        lax.fori_loop(0, nchunk // 2 - 1, _pair, None)

        jlast = nchunk - 2
        for b, (buf, sem) in enumerate(bufs):
            pltpu.make_async_copy(hs_sh.at[si.at[jlast + b]], buf, sem).wait()
            pltpu.async_copy(buf, acc.at[di.at[jlast + b]], semS, add=True)

        def _drain(j, _):
            pltpu.make_async_copy(rowsA, acc.at[di.at[j]], semS).wait()
            return _
        lax.fori_loop(0, nchunk, _drain, None)

        plsc.subcore_barrier()
        for q in range(RPT // CH):
            pltpu.sync_copy(acc.at[pl.ds(s * RPT + q * CH, CH)], bounce)
            pltpu.sync_copy(bounce,
                            o_hbm.at[c].at[pl.ds(s * RPT + q * CH, CH)])

    return seg_kernel


_B = 1000


def _dinv_pair(degp_blk):
    deg1 = jnp.sum(degp_blk[0], axis=1).reshape(_B, 1) + 1.0
    deg2 = jnp.sum(degp_blk[1], axis=1).reshape(_B, 1) + 1.0
    return lax.rsqrt(deg1), lax.rsqrt(deg2)


def _deg_spec():
    return pl.BlockSpec((2, _B, NW), lambda i: (0, i, 0))


def _stage1_body(x_ref, w11_ref, w21_ref, degp_ref, hs11_ref, hs21_ref):
    dinv1, dinv2 = _dinv_pair(degp_ref[...])
    x = x_ref[...]
    hs11_ref[...] = dinv1 * jnp.dot(x, w11_ref[...],
                                    preferred_element_type=jnp.float32)
    hs21_ref[...] = dinv2 * jnp.dot(x, w21_ref[...],
                                    preferred_element_type=jnp.float32)


def _elu(v):
    return jnp.where(v > 0, v, jnp.exp(jnp.minimum(v, 0.0)) - 1.0)


def _stage2_body(a11_ref, a21_ref, hs11_ref, hs21_ref, degp_ref,
                 b11_ref, b21_ref, wg1a_ref, wg1b_ref, bg1_ref,
                 w12_ref, w22_ref, hs12_ref, hs22_ref):
    dinv1, dinv2 = _dinv_pair(degp_ref[...])
    a11 = a11_ref[0] + a11_ref[1]
    a21 = a21_ref[0] + a21_ref[1]
    s11 = _elu(dinv1 * (a11 + hs11_ref[...]) + b11_ref[...])
    s21 = _elu(dinv2 * (a21 + hs21_ref[...]) + b21_ref[...])
    z = (jnp.sum(s11 * wg1a_ref[...], axis=1, keepdims=True)
         + jnp.sum(s21 * wg1b_ref[...], axis=1, keepdims=True) + bg1_ref[...])
    g = jax.nn.sigmoid(z)
    mid = g * s11 + (1.0 - g) * s21
    hs12_ref[...] = dinv1 * jnp.dot(mid, w12_ref[...],
                                    preferred_element_type=jnp.float32)
    hs22_ref[...] = dinv2 * jnp.dot(mid, w22_ref[...],
                                    preferred_element_type=jnp.float32)


def _stage3_body(a12_ref, a22_ref, hs12_ref, hs22_ref, degp_ref,
                 b12_ref, b22_ref, wg2a_ref, wg2b_ref, bg2_ref,
                 wt_ref, bt_ref, out_ref):
    dinv1, dinv2 = _dinv_pair(degp_ref[...])
    s12 = dinv1 * (a12_ref[0] + a12_ref[1] + hs12_ref[...]) + b12_ref[...]
    s22 = dinv2 * (a22_ref[0] + a22_ref[1] + hs22_ref[...]) + b22_ref[...]
    z = (jnp.sum(s12 * wg2a_ref[...], axis=1, keepdims=True)
         + jnp.sum(s22 * wg2b_ref[...], axis=1, keepdims=True) + bg2_ref[...])
    g = jax.nn.sigmoid(z)
    o = g * s12 + (1.0 - g) * s22
    o = jnp.dot(o, wt_ref[...], preferred_element_type=jnp.float32) + bt_ref[...]
    m = jnp.max(o, axis=1, keepdims=True)
    e = o - m
    out_ref[...] = e - jnp.log(jnp.sum(jnp.exp(e), axis=1, keepdims=True))


def _full(shape):
    nd = len(shape)
    return pl.BlockSpec(shape, lambda i, _nd=nd: (0,) * nd)


def _rows(width):
    return pl.BlockSpec((_B, width), lambda i: (i, 0))


def _acc_spec():
    return pl.BlockSpec((NC, _B, D), lambda i: (0, i, 0))


def kernel(node_feature, one_adj_list, two_adj_list, W11, b11, W12, b12,
           W21, b21, W22, b22, Wg1, bg1, Wg2, bg2, Wt, bt):
    n, d_in = node_feature.shape
    e = one_adj_list.shape[1]
    ept = -(-e // (NW * 2 * CH)) * 2 * CH
    nchunk = ept // CH
    pad = NW * ept - e

    def prep(adj):
        src = jnp.concatenate([adj[0], jnp.zeros((pad,), jnp.int32)])
        dst = jnp.concatenate([adj[1], jnp.full((pad,), DUMMY, jnp.int32)])
        return src.reshape(NW, nchunk, CH), dst.reshape(NW, nchunk, CH)

    s1r, d1r = prep(one_adj_list)
    s2r, d2r = prep(two_adj_list)

    deg_parts = _make_deg_kernel(nchunk)(d1r, d2r)
    degp = jnp.swapaxes(deg_parts, 1, 2)

    grid = n // _B
    hs_t = jax.ShapeDtypeStruct((n, D), jnp.float32)

    hs11, hs21 = pl.pallas_call(
        _stage1_body,
        grid=(grid,),
        in_specs=[_rows(d_in), _full((d_in, D)), _full((d_in, D)), _deg_spec()],
        out_specs=[_rows(D), _rows(D)],
        out_shape=[hs_t, hs_t],
    )(node_feature, W11, W21, degp)

    seg = _make_seg_kernel(nchunk)
    a11 = seg(hs11, s1r, d1r)
    a21 = seg(hs21, s2r, d2r)

    b11r = b11.reshape(1, D)
    b21r = b21.reshape(1, D)
    wg1a = Wg1[:D, 0].reshape(1, D)
    wg1b = Wg1[D:, 0].reshape(1, D)
    bg1r = bg1.reshape(1, 1)

    hs12, hs22 = pl.pallas_call(
        _stage2_body,
        grid=(grid,),
        in_specs=[_acc_spec(), _acc_spec(), _rows(D), _rows(D), _deg_spec(),
                  _full((1, D)), _full((1, D)), _full((1, D)), _full((1, D)),
                  _full((1, 1)), _full((D, D)), _full((D, D))],
        out_specs=[_rows(D), _rows(D)],
        out_shape=[hs_t, hs_t],
    )(a11, a21, hs11, hs21, degp, b11r, b21r, wg1a, wg1b, bg1r, W12, W22)

    a12 = seg(hs12, s1r, d1r)
    a22 = seg(hs22, s2r, d2r)

    b12r = b12.reshape(1, D)
    b22r = b22.reshape(1, D)
    wg2a = Wg2[:D, 0].reshape(1, D)
    wg2b = Wg2[D:, 0].reshape(1, D)
    bg2r = bg2.reshape(1, 1)
    btr = bt.reshape(1, D)

    out = pl.pallas_call(
        _stage3_body,
        grid=(grid,),
        in_specs=[_acc_spec(), _acc_spec(), _rows(D), _rows(D), _deg_spec(),
                  _full((1, D)), _full((1, D)), _full((1, D)), _full((1, D)),
                  _full((1, 1)), _full((D, D)), _full((1, D))],
        out_specs=_rows(D),
        out_shape=jax.ShapeDtypeStruct((n, D), jnp.float32),
    )(a12, a22, hs12, hs22, degp, b12r, b22r, wg2a, wg2b, bg2r, Wt, btr)

    return out
